# Initial kernel scaffold; baseline (speedup 1.0000x reference)
import jax
import jax.numpy as jnp
from jax.experimental import pallas as pl


def kernel(attrs_scaled_stack, thr_norm_vec, node_of_pixel):
    raise NotImplementedError("write your pallas kernel here")


# speedup vs baseline: 202.7068x; 202.7068x over previous
import functools

import jax
import jax.numpy as jnp
from jax import lax
from jax._src import config as _jax_config
from jax.experimental import pallas as pl
from jax.experimental.pallas import tpu as pltpu
from jax.experimental.pallas import tpu_sc as plsc

N_NODES = 2097152
N_PIXELS = 4194304

NC = 2
NS = 16
NW = NC * NS

PW = N_PIXELS // NW
CHUNK = 8192
NCHUNK = PW // CHUNK


def _sigmoid_body(thr_ref, a_ref, o_ref):
    a0 = a_ref[0]
    a1 = a_ref[1]
    d = jnp.abs(a0 - thr_ref[0]) * jnp.abs(a1 - thr_ref[1])
    s = jnp.clip(d * -1000.0, -12.0, 12.0)
    o_ref[...] = jax.nn.sigmoid(s)


def _sigmoid_table(attrs, thr):
    blk = 262144
    grid = (N_NODES // blk,)
    return pl.pallas_call(
        _sigmoid_body,
        grid=grid,
        in_specs=[
            pl.BlockSpec((2,), lambda i: (0,), memory_space=pltpu.SMEM),
            pl.BlockSpec((2, blk), lambda i: (0, i)),
        ],
        out_specs=pl.BlockSpec((blk,), lambda i: (i,)),
        out_shape=jax.ShapeDtypeStruct((N_NODES,), jnp.float32),
    )(thr, attrs)


def _gather_body(table_hbm, idx_hbm, out_hbm, idx_v, rows_v, sem):
    wid = lax.axis_index("s") * NC + lax.axis_index("c")

    base0 = wid * jnp.int32(PW)
    for k in range(NCHUNK):
        base = base0 + jnp.int32(k * CHUNK)
        pltpu.sync_copy(idx_hbm.at[pl.ds(base, CHUNK)], idx_v)
        pltpu.async_copy(table_hbm.at[idx_v], rows_v, sem).wait()
        pltpu.sync_copy(rows_v, out_hbm.at[pl.ds(base, CHUNK)])


_sc_gather = functools.partial(
    pl.kernel,
    mesh=plsc.VectorSubcoreMesh(core_axis_name="c", subcore_axis_name="s"),
    out_type=jax.ShapeDtypeStruct((N_PIXELS,), jnp.float32),
    scratch_types=[
        pltpu.VMEM((CHUNK,), jnp.int32),
        pltpu.VMEM((CHUNK,), jnp.float32),
        pltpu.SemaphoreType.DMA,
    ],
)(_gather_body)


def kernel(attrs_scaled_stack, thr_norm_vec, node_of_pixel):
    with _jax_config.enable_x64(False):
        idx32 = node_of_pixel.astype(jnp.int32)
        table = _sigmoid_table(attrs_scaled_stack, thr_norm_vec)
        out = _sc_gather(table, idx32)
    return out


# speedup vs baseline: 213.8114x; 1.0548x over previous
import functools

import jax
import jax.numpy as jnp
from jax import lax
from jax._src import config as _jax_config
from jax.experimental import pallas as pl
from jax.experimental.pallas import tpu as pltpu
from jax.experimental.pallas import tpu_sc as plsc

N_NODES = 2097152
N_PIXELS = 4194304

NC = 2
NS = 16
NW = NC * NS

PW = N_PIXELS // NW
CHUNK = 8192
NCHUNK = PW // CHUNK


def _sigmoid_body(thr_ref, a_ref, o_ref):
    a0 = a_ref[0]
    a1 = a_ref[1]
    d = jnp.abs(a0 - thr_ref[0]) * jnp.abs(a1 - thr_ref[1])
    s = jnp.clip(d * -1000.0, -12.0, 12.0)
    o_ref[...] = jax.nn.sigmoid(s)


def _sigmoid_table(attrs, thr):
    blk = 262144
    grid = (N_NODES // blk,)
    return pl.pallas_call(
        _sigmoid_body,
        grid=grid,
        in_specs=[
            pl.BlockSpec((2,), lambda i: (0,), memory_space=pltpu.SMEM),
            pl.BlockSpec((2, blk), lambda i: (0, i)),
        ],
        out_specs=pl.BlockSpec((blk,), lambda i: (i,)),
        out_shape=jax.ShapeDtypeStruct((N_NODES,), jnp.float32),
    )(thr, attrs)


def _gather_body(table_hbm, idx_hbm, out_hbm,
                 idx0, idx1, rows0, rows1,
                 isem0, isem1, gsem, osem0, osem1):
    wid = lax.axis_index("s") * NC + lax.axis_index("c")
    base0 = wid * jnp.int32(PW)

    idx_v = (idx0, idx1)
    rows_v = (rows0, rows1)
    isem = (isem0, isem1)
    osem = (osem0, osem1)

    def chunk_base(k):
        return base0 + jnp.int32(k * CHUNK)

    pltpu.async_copy(idx_hbm.at[pl.ds(chunk_base(0), CHUNK)], idx_v[0], isem[0])

    for k in range(NCHUNK):
        b = k % 2
        nb = (k + 1) % 2
        if k + 1 < NCHUNK:
            pltpu.async_copy(
                idx_hbm.at[pl.ds(chunk_base(k + 1), CHUNK)], idx_v[nb], isem[nb])
        pltpu.make_async_copy(idx_hbm.at[pl.ds(chunk_base(k), CHUNK)],
                              idx_v[b], isem[b]).wait()
        if k >= 2:
            pltpu.make_async_copy(rows_v[b],
                                  out_hbm.at[pl.ds(chunk_base(k - 2), CHUNK)],
                                  osem[b]).wait()
        pltpu.async_copy(table_hbm.at[idx_v[b]], rows_v[b], gsem).wait()
        pltpu.async_copy(rows_v[b], out_hbm.at[pl.ds(chunk_base(k), CHUNK)],
                         osem[b])

    for k in (NCHUNK - 2, NCHUNK - 1):
        b = k % 2
        pltpu.make_async_copy(rows_v[b],
                              out_hbm.at[pl.ds(chunk_base(k), CHUNK)],
                              osem[b]).wait()


_sc_gather = functools.partial(
    pl.kernel,
    mesh=plsc.VectorSubcoreMesh(core_axis_name="c", subcore_axis_name="s"),
    out_type=jax.ShapeDtypeStruct((N_PIXELS,), jnp.float32),
    scratch_types=[
        pltpu.VMEM((CHUNK,), jnp.int32),
        pltpu.VMEM((CHUNK,), jnp.int32),
        pltpu.VMEM((CHUNK,), jnp.float32),
        pltpu.VMEM((CHUNK,), jnp.float32),
        pltpu.SemaphoreType.DMA,
        pltpu.SemaphoreType.DMA,
        pltpu.SemaphoreType.DMA,
        pltpu.SemaphoreType.DMA,
        pltpu.SemaphoreType.DMA,
    ],
)(_gather_body)


def kernel(attrs_scaled_stack, thr_norm_vec, node_of_pixel):
    with _jax_config.enable_x64(False):
        idx32 = node_of_pixel.astype(jnp.int32)
        table = _sigmoid_table(attrs_scaled_stack, thr_norm_vec)
        out = _sc_gather(table, idx32)
    return out
